# baseline (speedup 1.0000x reference)
import functools

import jax
import jax.numpy as jnp
from jax import lax
from jax.experimental import pallas as pl
from jax.experimental.pallas import tpu as pltpu
from jax.experimental.pallas import tpu_sc as plsc

DIMS = (64, 64, 64)
FEAT = 64
BATCH = 16384
NROWS = DIMS[0] * DIMS[1] * DIMS[2]

NUM_CORES = 2
NUM_SUBCORES = 16
LANES = 16
NUM_WORKERS = NUM_CORES * NUM_SUBCORES
BPW = BATCH // NUM_WORKERS
KBATCH = 256
NBATCH = BPW // KBATCH

_mesh = plsc.VectorSubcoreMesh(core_axis_name="c", subcore_axis_name="s")


@functools.partial(
    pl.kernel,
    mesh=_mesh,
    out_type=jax.ShapeDtypeStruct((BATCH, FEAT), jnp.float32),
    scratch_types=[
        pltpu.VMEM((3 * BPW,), jnp.int32),
        pltpu.VMEM((BPW, FEAT), jnp.float32),
        pltpu.SemaphoreType.DMA,
    ],
)
def _lookup(idx_hbm, tab_hbm, out_hbm, raw_v, rows_v, sem):
    wid = lax.axis_index("s") * NUM_CORES + lax.axis_index("c")
    base = pl.multiple_of(wid * BPW, BPW)

    stage = [
        pltpu.async_copy(idx_hbm.at[pl.ds(c * BATCH + base, BPW)],
                         raw_v.at[pl.ds(c * BPW, BPW)], sem)
        for c in range(3)
    ]
    for cp in stage:
        cp.wait()

    @pl.loop(0, NBATCH)
    def _batch(g):
        r0 = g * KBATCH
        for h in range(KBATCH // LANES):
            o16 = r0 + h * LANES
            i0 = raw_v[pl.ds(o16, LANES)]
            i1 = raw_v[pl.ds(BPW + o16, LANES)]
            i2 = raw_v[pl.ds(2 * BPW + o16, LANES)]
            fv = i0 * (DIMS[1] * DIMS[2]) + i1 * DIMS[2] + i2
            for l in range(LANES):
                pltpu.async_copy(
                    tab_hbm.at[fv[l]],
                    rows_v.at[r0 + h * LANES + l],
                    sem,
                )
    pltpu.make_async_copy(
        tab_hbm.at[pl.ds(0, BPW)],
        rows_v,
        sem,
    ).wait()

    pltpu.sync_copy(rows_v, out_hbm.at[pl.ds(base, BPW)])


def kernel(indices, table):
    idx_cols = indices.astype(jnp.int32).T.reshape(-1)
    tab2d = table.reshape(NROWS, FEAT)
    return _lookup(idx_cols, tab2d)


# speedup vs baseline: 1.0438x; 1.0438x over previous
import functools

import jax
import jax.numpy as jnp
from jax import lax
from jax.experimental import pallas as pl
from jax.experimental.pallas import tpu as pltpu
from jax.experimental.pallas import tpu_sc as plsc

DIMS = (64, 64, 64)
FEAT = 64
BATCH = 16384
NROWS = DIMS[0] * DIMS[1] * DIMS[2]

NUM_CORES = 2
NUM_SUBCORES = 16
LANES = 16
NUM_WORKERS = NUM_CORES * NUM_SUBCORES
BPW = BATCH // NUM_WORKERS
KBATCH = 64
NBATCH = BPW // KBATCH

_mesh = plsc.VectorSubcoreMesh(core_axis_name="c", subcore_axis_name="s")


@functools.partial(
    pl.kernel,
    mesh=_mesh,
    out_type=jax.ShapeDtypeStruct((BATCH, FEAT), jnp.float32),
    scratch_types=[
        pltpu.VMEM((3 * BPW,), jnp.int32),
        pltpu.VMEM((BPW, FEAT), jnp.float32),
        pltpu.SemaphoreType.DMA,
    ],
)
def _lookup(idx_hbm, tab_hbm, out_hbm, raw_v, rows_v, sem):
    wid = lax.axis_index("s") * NUM_CORES + lax.axis_index("c")
    base = pl.multiple_of(wid * BPW, BPW)

    stage = [
        pltpu.async_copy(idx_hbm.at[pl.ds(c * BATCH + base, BPW)],
                         raw_v.at[pl.ds(c * BPW, BPW)], sem)
        for c in range(3)
    ]
    for cp in stage:
        cp.wait()

    @pl.loop(0, NBATCH)
    def _batch(g):
        r0 = g * KBATCH
        for h in range(KBATCH // LANES):
            o16 = r0 + h * LANES
            i0 = raw_v[pl.ds(o16, LANES)]
            i1 = raw_v[pl.ds(BPW + o16, LANES)]
            i2 = raw_v[pl.ds(2 * BPW + o16, LANES)]
            fv = i0 * (DIMS[1] * DIMS[2]) + i1 * DIMS[2] + i2
            for l in range(LANES):
                pltpu.async_copy(
                    tab_hbm.at[fv[l]],
                    rows_v.at[r0 + h * LANES + l],
                    sem,
                )
    pltpu.make_async_copy(
        tab_hbm.at[pl.ds(0, BPW)],
        rows_v,
        sem,
    ).wait()

    pltpu.sync_copy(rows_v, out_hbm.at[pl.ds(base, BPW)])


def kernel(indices, table):
    idx_cols = indices.astype(jnp.int32).T.reshape(-1)
    tab2d = table.reshape(NROWS, FEAT)
    return _lookup(idx_cols, tab2d)


# speedup vs baseline: 1.0617x; 1.0171x over previous
import functools

import jax
import jax.numpy as jnp
from jax import lax
from jax.experimental import pallas as pl
from jax.experimental.pallas import tpu as pltpu
from jax.experimental.pallas import tpu_sc as plsc

DIMS = (64, 64, 64)
FEAT = 64
BATCH = 16384
NROWS = DIMS[0] * DIMS[1] * DIMS[2]

NUM_CORES = 2
NUM_SUBCORES = 16
LANES = 16
NUM_WORKERS = NUM_CORES * NUM_SUBCORES
BPW = BATCH // NUM_WORKERS
KBATCH = 32
NBATCH = BPW // KBATCH

_mesh = plsc.VectorSubcoreMesh(core_axis_name="c", subcore_axis_name="s")


@functools.partial(
    pl.kernel,
    mesh=_mesh,
    out_type=jax.ShapeDtypeStruct((BATCH, FEAT), jnp.float32),
    scratch_types=[
        pltpu.VMEM((3 * BPW,), jnp.int32),
        pltpu.VMEM((BPW, FEAT), jnp.float32),
        pltpu.SemaphoreType.DMA,
    ],
)
def _lookup(idx_hbm, tab_hbm, out_hbm, raw_v, rows_v, sem):
    wid = lax.axis_index("s") * NUM_CORES + lax.axis_index("c")
    base = pl.multiple_of(wid * BPW, BPW)

    stage = [
        pltpu.async_copy(idx_hbm.at[pl.ds(c * BATCH + base, BPW)],
                         raw_v.at[pl.ds(c * BPW, BPW)], sem)
        for c in range(3)
    ]
    for cp in stage:
        cp.wait()

    @pl.loop(0, NBATCH)
    def _batch(g):
        r0 = g * KBATCH
        for h in range(KBATCH // LANES):
            o16 = r0 + h * LANES
            i0 = raw_v[pl.ds(o16, LANES)]
            i1 = raw_v[pl.ds(BPW + o16, LANES)]
            i2 = raw_v[pl.ds(2 * BPW + o16, LANES)]
            fv = i0 * (DIMS[1] * DIMS[2]) + i1 * DIMS[2] + i2
            for l in range(LANES):
                pltpu.async_copy(
                    tab_hbm.at[fv[l]],
                    rows_v.at[r0 + h * LANES + l],
                    sem,
                )
    pltpu.make_async_copy(
        tab_hbm.at[pl.ds(0, BPW)],
        rows_v,
        sem,
    ).wait()

    pltpu.sync_copy(rows_v, out_hbm.at[pl.ds(base, BPW)])


def kernel(indices, table):
    idx_cols = indices.astype(jnp.int32).T.reshape(-1)
    tab2d = table.reshape(NROWS, FEAT)
    return _lookup(idx_cols, tab2d)


# speedup vs baseline: 1.0618x; 1.0001x over previous
import functools

import jax
import jax.numpy as jnp
from jax import lax
from jax.experimental import pallas as pl
from jax.experimental.pallas import tpu as pltpu
from jax.experimental.pallas import tpu_sc as plsc

DIMS = (64, 64, 64)
FEAT = 64
BATCH = 16384
NROWS = DIMS[0] * DIMS[1] * DIMS[2]

NUM_CORES = 2
NUM_SUBCORES = 16
LANES = 16
NUM_WORKERS = NUM_CORES * NUM_SUBCORES
BPW = BATCH // NUM_WORKERS
KBATCH = 16
NBATCH = BPW // KBATCH

_mesh = plsc.VectorSubcoreMesh(core_axis_name="c", subcore_axis_name="s")


@functools.partial(
    pl.kernel,
    mesh=_mesh,
    out_type=jax.ShapeDtypeStruct((BATCH, FEAT), jnp.float32),
    scratch_types=[
        pltpu.VMEM((3 * BPW,), jnp.int32),
        pltpu.VMEM((BPW, FEAT), jnp.float32),
        pltpu.SemaphoreType.DMA,
    ],
)
def _lookup(idx_hbm, tab_hbm, out_hbm, raw_v, rows_v, sem):
    wid = lax.axis_index("s") * NUM_CORES + lax.axis_index("c")
    base = pl.multiple_of(wid * BPW, BPW)

    stage = [
        pltpu.async_copy(idx_hbm.at[pl.ds(c * BATCH + base, BPW)],
                         raw_v.at[pl.ds(c * BPW, BPW)], sem)
        for c in range(3)
    ]
    for cp in stage:
        cp.wait()

    @pl.loop(0, NBATCH)
    def _batch(g):
        r0 = g * KBATCH
        for h in range(KBATCH // LANES):
            o16 = r0 + h * LANES
            i0 = raw_v[pl.ds(o16, LANES)]
            i1 = raw_v[pl.ds(BPW + o16, LANES)]
            i2 = raw_v[pl.ds(2 * BPW + o16, LANES)]
            fv = i0 * (DIMS[1] * DIMS[2]) + i1 * DIMS[2] + i2
            for l in range(LANES):
                pltpu.async_copy(
                    tab_hbm.at[fv[l]],
                    rows_v.at[r0 + h * LANES + l],
                    sem,
                )
    pltpu.make_async_copy(
        tab_hbm.at[pl.ds(0, BPW)],
        rows_v,
        sem,
    ).wait()

    pltpu.sync_copy(rows_v, out_hbm.at[pl.ds(base, BPW)])


def kernel(indices, table):
    idx_cols = indices.astype(jnp.int32).T.reshape(-1)
    tab2d = table.reshape(NROWS, FEAT)
    return _lookup(idx_cols, tab2d)
